# baseline (speedup 1.0000x reference)
import functools

import jax
import jax.numpy as jnp
from jax import lax
from jax.experimental import pallas as pl
from jax.experimental.pallas import tpu as pltpu
from jax.experimental.pallas import tpu_sc as plsc

N = 4096
F = 784
DM = 256
TOPK = 11
RB = 256
NB = N // RB
INV_SQRT_K = 11.0 ** -0.5
BIG = 3.0e38

NW = 32
PW = N // NW
CH = 8
NCH = PW // CH
LANES = 16


def _k1_body(x_blk_ref, x_all_ref, theta_ref, h_ref, cnt_ref, y_ref,
             idx_ref):
    i = pl.program_id(0)
    x = x_all_ref[...]
    xb = x_blk_ref[...]
    sq_all = jnp.sum(x * x, axis=1)
    sq_b = jnp.sum(xb * xb, axis=1, keepdims=True)
    c = lax.dot_general(xb, x, (((1,), (1,)), ((), ())),
                        preferred_element_type=jnp.float32)
    dist = jnp.abs(sq_b + sq_all[None, :] - 2.0 * c)
    col = lax.broadcasted_iota(jnp.int32, (RB, N), 1)
    d = dist
    picks = []
    for _ in range(TOPK):
        m = jnp.min(d, axis=1, keepdims=True)
        idx = jnp.min(jnp.where(d == m, col, N), axis=1, keepdims=True)
        picks.append(idx)
        d = jnp.where(col == idx, BIG, d)
    idx_ref[...] = jnp.concatenate(picks, axis=1)
    onehots = jnp.where(d == BIG, 1.0, 0.0)
    hb = onehots.T
    h_ref[...] = hb
    cnt = jnp.sum(hb, axis=1, keepdims=True)

    @pl.when(i == 0)
    def _():
        cnt_ref[...] = jnp.zeros_like(cnt_ref)

    cnt_ref[...] += cnt
    y_ref[...] = jnp.dot(xb, theta_ref[...],
                         preferred_element_type=jnp.float32)


def _k2a_body(y_ref, cnt_ref, dvy_ref):
    dvy_ref[...] = y_ref[...] * lax.rsqrt(cnt_ref[...])


def _k3_body(h_ref, xo_ref, cnt_ref, e_ref):
    acc = jnp.dot(h_ref[...], xo_ref[...],
                  preferred_element_type=jnp.float32)
    dv = lax.rsqrt(cnt_ref[...])
    e_ref[...] = acc * dv * INV_SQRT_K


def _make_k1():
    return pl.pallas_call(
        _k1_body,
        grid=(NB,),
        in_specs=[
            pl.BlockSpec((RB, F), lambda i: (i, 0)),
            pl.BlockSpec((N, F), lambda i: (0, 0)),
            pl.BlockSpec((F, DM), lambda i: (0, 0)),
        ],
        out_specs=[
            pl.BlockSpec((N, RB), lambda i: (0, i)),
            pl.BlockSpec((N, 1), lambda i: (0, 0)),
            pl.BlockSpec((RB, DM), lambda i: (i, 0)),
            pl.BlockSpec((RB, TOPK), lambda i: (i, 0)),
        ],
        out_shape=[
            jax.ShapeDtypeStruct((N, N), jnp.float32),
            jax.ShapeDtypeStruct((N, 1), jnp.float32),
            jax.ShapeDtypeStruct((N, DM), jnp.float32),
            jax.ShapeDtypeStruct((N, TOPK), jnp.int32),
        ],
    )


def _make_k2a():
    return pl.pallas_call(
        _k2a_body,
        grid=(1,),
        in_specs=[
            pl.BlockSpec((N, DM), lambda i: (0, 0)),
            pl.BlockSpec((N, 1), lambda i: (0, 0)),
        ],
        out_specs=pl.BlockSpec((N, DM), lambda i: (0, 0)),
        out_shape=jax.ShapeDtypeStruct((N, DM), jnp.float32),
    )


def _sc_xo_kernel():
    mesh = plsc.VectorSubcoreMesh(core_axis_name="c", subcore_axis_name="s")

    @functools.partial(
        pl.kernel,
        mesh=mesh,
        out_type=jax.ShapeDtypeStruct((N, DM), jnp.float32),
        scratch_types=[
            pltpu.VMEM((NCH, CH * TOPK), jnp.int32),
            pltpu.VMEM((CH * TOPK, DM), jnp.float32),
            pltpu.VMEM((CH * TOPK, DM), jnp.float32),
            pltpu.VMEM((PW, DM), jnp.float32),
            pltpu.SemaphoreType.DMA,
            pltpu.SemaphoreType.DMA,
        ],
    )
    def k(dvy_hbm, idx3_hbm, xo_hbm, idxv, rows_a, rows_b, xov,
          sem_a, sem_b):
        cc = lax.axis_index("c")
        ss = lax.axis_index("s")
        wid = ss * 2 + cc
        pltpu.sync_copy(idx3_hbm.at[wid], idxv)

        def issue(ch, buf, sem):
            pltpu.make_async_copy(
                dvy_hbm.at[idxv.at[ch]], buf, sem).start()

        def drain(ch, buf, sem):
            pltpu.make_async_copy(
                dvy_hbm.at[idxv.at[ch]], buf, sem).wait()

        def sum_chunk(ch, buf):
            def e_body(j, _):
                r0 = j * TOPK
                for cblk in range(DM // LANES):
                    sl = pl.ds(cblk * LANES, LANES)
                    acc = buf[r0, sl]
                    for kk in range(1, TOPK):
                        acc = acc + buf[r0 + kk, sl]
                    xov[ch * CH + j, sl] = acc * INV_SQRT_K
                return 0

            lax.fori_loop(0, CH, e_body, 0)

        issue(0, rows_a, sem_a)

        def pair_body(i, _):
            ch_a = i * 2
            ch_b = ch_a + 1
            issue(ch_b, rows_b, sem_b)
            drain(ch_a, rows_a, sem_a)
            sum_chunk(ch_a, rows_a)

            @pl.when(i < (NCH // 2) - 1)
            def _():
                issue(ch_a + 2, rows_a, sem_a)

            drain(ch_b, rows_b, sem_b)
            sum_chunk(ch_b, rows_b)
            return 0

        lax.fori_loop(0, NCH // 2, pair_body, 0)
        pltpu.sync_copy(xov, xo_hbm.at[pl.ds(wid * PW, PW)])

    return k


def _make_k3():
    return pl.pallas_call(
        _k3_body,
        grid=(NB,),
        in_specs=[
            pl.BlockSpec((RB, N), lambda r: (r, 0)),
            pl.BlockSpec((N, DM), lambda r: (0, 0)),
            pl.BlockSpec((RB, 1), lambda r: (r, 0)),
        ],
        out_specs=pl.BlockSpec((RB, DM), lambda r: (r, 0)),
        out_shape=jax.ShapeDtypeStruct((N, DM), jnp.float32),
    )


def kernel(X, theta):
    X = X.reshape(-1, F)
    H, cnt, Y, idx = _make_k1()(X, X, theta)
    dvy = _make_k2a()(Y, cnt)
    idx3 = idx.reshape(NW, NCH, CH * TOPK)
    Xo = _sc_xo_kernel()(dvy, idx3)
    E = _make_k3()(H, Xo, cnt)
    return (Xo, E, H)


# speedup vs baseline: 1.1440x; 1.1440x over previous
import jax
import jax.numpy as jnp
from jax import lax
from jax.experimental import pallas as pl
from jax.experimental.pallas import tpu as pltpu

N = 4096
F = 784
DM = 256
TOPK = 11
RB = 256
NB = N // RB
INV_SQRT_K = 11.0 ** -0.5
BIG = 3.0e38


def _k1_body(x_blk_ref, x_all_ref, theta_ref, sqb_ref, sqt_ref,
             h_ref, cnt_ref, y_ref):
    i = pl.program_id(0)
    x = x_all_ref[...]
    xb = x_blk_ref[...]
    c = lax.dot_general(xb, x, (((1,), (1,)), ((), ())),
                        preferred_element_type=jnp.float32)
    dist = jnp.abs(sqb_ref[...] + sqt_ref[...] - 2.0 * c)
    col = lax.broadcasted_iota(jnp.int32, (RB, N), 1)
    d = dist
    for _ in range(TOPK):
        m = jnp.min(d, axis=1, keepdims=True)
        idx = jnp.min(jnp.where(d == m, col, N), axis=1, keepdims=True)
        d = jnp.where(col == idx, BIG, d)
    onehots = jnp.where(d == BIG, 1.0, 0.0)
    hb = onehots.T
    h_ref[...] = hb
    cnt = jnp.sum(hb, axis=1, keepdims=True)

    @pl.when(i == 0)
    def _():
        cnt_ref[...] = jnp.zeros_like(cnt_ref)

    cnt_ref[...] += cnt
    y_ref[...] = jnp.dot(xb, theta_ref[...],
                         preferred_element_type=jnp.float32)


def _k2_body(h_ref, y_ref, cnt_ref, xo_ref):
    dv = lax.rsqrt(cnt_ref[...])
    dvy = y_ref[...] * dv
    xo = lax.dot_general(h_ref[...], dvy, (((0,), (0,)), ((), ())),
                         preferred_element_type=jnp.float32)
    xo_ref[...] = xo * INV_SQRT_K


def _k3_body(h_ref, xo_ref, cnt_ref, e_ref):
    acc = jnp.dot(h_ref[...], xo_ref[...],
                  preferred_element_type=jnp.float32)
    dv = lax.rsqrt(cnt_ref[...])
    e_ref[...] = acc * dv * INV_SQRT_K


def _make_k1(interpret=False):
    return pl.pallas_call(
        _k1_body,
        grid=(NB,),
        in_specs=[
            pl.BlockSpec((RB, F), lambda i: (i, 0)),
            pl.BlockSpec((N, F), lambda i: (0, 0)),
            pl.BlockSpec((F, DM), lambda i: (0, 0)),
            pl.BlockSpec((RB, 1), lambda i: (i, 0)),
            pl.BlockSpec((1, N), lambda i: (0, 0)),
        ],
        out_specs=[
            pl.BlockSpec((N, RB), lambda i: (0, i)),
            pl.BlockSpec((N, 1), lambda i: (0, 0)),
            pl.BlockSpec((RB, DM), lambda i: (i, 0)),
        ],
        out_shape=[
            jax.ShapeDtypeStruct((N, N), jnp.float32),
            jax.ShapeDtypeStruct((N, 1), jnp.float32),
            jax.ShapeDtypeStruct((N, DM), jnp.float32),
        ],
        interpret=interpret,
    )


def _make_k2(interpret=False):
    return pl.pallas_call(
        _k2_body,
        grid=(NB,),
        in_specs=[
            pl.BlockSpec((N, RB), lambda e: (0, e)),
            pl.BlockSpec((N, DM), lambda e: (0, 0)),
            pl.BlockSpec((N, 1), lambda e: (0, 0)),
        ],
        out_specs=pl.BlockSpec((RB, DM), lambda e: (e, 0)),
        out_shape=jax.ShapeDtypeStruct((N, DM), jnp.float32),
        interpret=interpret,
    )


def _make_k3(interpret=False):
    return pl.pallas_call(
        _k3_body,
        grid=(NB,),
        in_specs=[
            pl.BlockSpec((RB, N), lambda r: (r, 0)),
            pl.BlockSpec((N, DM), lambda r: (0, 0)),
            pl.BlockSpec((RB, 1), lambda r: (r, 0)),
        ],
        out_specs=pl.BlockSpec((RB, DM), lambda r: (r, 0)),
        out_shape=jax.ShapeDtypeStruct((N, DM), jnp.float32),
        interpret=interpret,
    )


def kernel(X, theta, interpret=False):
    X = X.reshape(-1, F)
    sq = jnp.sum(X ** 2, axis=1)
    H, cnt, Y = _make_k1(interpret)(X, X, theta, sq.reshape(N, 1),
                                    sq.reshape(1, N))
    Xo = _make_k2(interpret)(H, Y, cnt)
    E = _make_k3(interpret)(H, Xo, cnt)
    return (Xo, E, H)
